# Initial kernel scaffold; baseline (speedup 1.0000x reference)
import functools

import jax
import jax.numpy as jnp
from jax import lax
from jax.experimental import pallas as pl
from jax.experimental.pallas import tpu as pltpu
from jax.experimental.pallas import tpu_sc as plsc

DIM = 1024
NUM_EXPERTS = 8
TOP_K = 2
TOKENS = 4096
TILE = 128
NUM_TILES = 72
NUM_SLOTS = NUM_TILES * TILE
SLOTS = TOKENS * TOP_K
NUM_WORKERS = 32
SLOTS_PER_WORKER = SLOTS // NUM_WORKERS
CHUNK = 64
EPS = 1e-6
OUT_TM = 512


def _lane_cumsum(a):
    n = a.shape[-1]
    d = 1
    while d < n:
        shifted = jnp.concatenate(
            [jnp.zeros(a.shape[:-1] + (d,), a.dtype), a[..., :-d]], axis=-1)
        a = a + shifted
        d *= 2
    return a


def _routing_body(x_ref, wr_ref, br_ref, pos_ref, w_ref, te_ref):
    x = x_ref[...]
    lt = lax.dot_general(wr_ref[...], x, (((1,), (1,)), ((), ())),
                         precision=lax.Precision.HIGHEST,
                         preferred_element_type=jnp.float32)
    lt = lt + br_ref[...]
    riota = lax.broadcasted_iota(jnp.int32, lt.shape, 0)
    m1 = jnp.max(lt, axis=0, keepdims=True)
    i1 = jnp.min(jnp.where(lt == m1, riota, NUM_EXPERTS), axis=0,
                 keepdims=True)
    sel1 = riota == i1
    lt2 = jnp.where(sel1, -1e30, lt)
    m2 = jnp.max(lt2, axis=0, keepdims=True)
    i2 = jnp.min(jnp.where(lt2 == m2, riota, NUM_EXPERTS), axis=0,
                 keepdims=True)
    sel2 = riota == i2
    w1 = jax.nn.sigmoid(m1 - m2)
    w_ref[...] = jnp.concatenate([w1, 1.0 - w1], axis=0)
    a1 = sel1.astype(jnp.int32)
    a2 = sel2.astype(jnp.int32)
    c1 = _lane_cumsum(a1)
    c2 = _lane_cumsum(a2)
    tot1 = c1[:, TOKENS - 1:]
    counts = tot1 + c2[:, TOKENS - 1:]
    ntiles = (counts + (TILE - 1)) >> 7
    inc = ntiles
    for d in (1, 2, 4):
        inc = inc + jnp.concatenate(
            [jnp.zeros((d, 1), jnp.int32), inc[:-d]], axis=0)
    toff = inc - ntiles
    base = toff * TILE
    p0 = base + (c1 - a1)
    p1 = base + tot1 + (c2 - a2)
    pos0 = jnp.sum(a1 * p0, axis=0, keepdims=True)
    pos1 = jnp.sum(a2 * p1, axis=0, keepdims=True)
    pos_ref[...] = jnp.concatenate([pos0, pos1], axis=0)
    jiota = lax.broadcasted_iota(jnp.int32, (NUM_EXPERTS, 128), 1)
    te_ref[...] = jnp.sum((jiota >= toff).astype(jnp.int32), axis=0,
                          keepdims=True) - 1


def _routing(x2d, Wr, br):
    return pl.pallas_call(
        _routing_body,
        out_shape=(
            jax.ShapeDtypeStruct((TOP_K, TOKENS), jnp.int32),
            jax.ShapeDtypeStruct((TOP_K, TOKENS), jnp.float32),
            jax.ShapeDtypeStruct((1, 128), jnp.int32),
        ),
    )(x2d, Wr, br.reshape(NUM_EXPERTS, 1))


_sc_mesh = plsc.VectorSubcoreMesh(core_axis_name="c", subcore_axis_name="s")


@functools.partial(
    pl.kernel,
    out_type=jax.ShapeDtypeStruct((NUM_SLOTS, DIM), jnp.float32),
    mesh=_sc_mesh,
    scratch_types=[
        pltpu.VMEM((CHUNK,), jnp.int32),
        pltpu.VMEM((CHUNK, DIM), jnp.float32),
        pltpu.SemaphoreType.DMA,
    ],
)
def _sc_dispatch(x_hbm, pos_hbm, xg_hbm, pos_v, rows_v, sem):
    wid = lax.axis_index("s") * 2 + lax.axis_index("c")

    @pl.loop(0, SLOTS_PER_WORKER // CHUNK)
    def _(ci):
        s_base = wid * SLOTS_PER_WORKER + ci * CHUNK
        t_base = lax.rem(s_base, TOKENS)
        pltpu.sync_copy(pos_hbm.at[pl.ds(s_base, CHUNK)], pos_v)
        pltpu.sync_copy(x_hbm.at[pl.ds(t_base, CHUNK)], rows_v)
        pltpu.async_copy(rows_v, xg_hbm.at[pos_v], sem).wait()


@functools.partial(
    pl.kernel,
    out_type=jax.ShapeDtypeStruct((SLOTS, DIM), jnp.float32),
    mesh=_sc_mesh,
    scratch_types=[
        pltpu.VMEM((CHUNK,), jnp.int32),
        pltpu.VMEM((CHUNK, DIM), jnp.float32),
        pltpu.SemaphoreType.DMA,
    ],
)
def _sc_combine(yg_hbm, pos_hbm, gc_hbm, pos_v, rows_v, sem):
    wid = lax.axis_index("s") * 2 + lax.axis_index("c")

    @pl.loop(0, SLOTS_PER_WORKER // CHUNK)
    def _(ci):
        s_base = wid * SLOTS_PER_WORKER + ci * CHUNK
        pltpu.sync_copy(pos_hbm.at[pl.ds(s_base, CHUNK)], pos_v)
        pltpu.async_copy(yg_hbm.at[pos_v], rows_v, sem).wait()
        pltpu.sync_copy(rows_v, gc_hbm.at[pl.ds(s_base, CHUNK)])


def _expert_body(te_ref, xg_ref, we_ref, be_ref, yg_ref):
    xb = xg_ref[...].astype(jnp.bfloat16)
    wb = we_ref[0].astype(jnp.bfloat16)
    h = lax.dot_general(xb, wb, (((1,), (1,)), ((), ())),
                        preferred_element_type=jnp.float32)
    h = h + be_ref[...]
    yg_ref[...] = h * jax.nn.sigmoid(h)


def _expert_matmuls(te, xg, We, be):
    grid_spec = pltpu.PrefetchScalarGridSpec(
        num_scalar_prefetch=1,
        grid=(NUM_TILES,),
        in_specs=[
            pl.BlockSpec((TILE, DIM), lambda j, te: (j, 0)),
            pl.BlockSpec((1, DIM, DIM), lambda j, te: (te[j], 0, 0)),
            pl.BlockSpec((1, DIM), lambda j, te: (te[j], 0)),
        ],
        out_specs=pl.BlockSpec((TILE, DIM), lambda j, te: (j, 0)),
    )
    return pl.pallas_call(
        _expert_body,
        grid_spec=grid_spec,
        out_shape=jax.ShapeDtypeStruct((NUM_SLOTS, DIM), jnp.float32),
    )(te, xg, We, be)


def _output_body(x_ref, gc_ref, wt_ref, wo_ref, bo_ref, g_ref, y_ref):
    comb = gc_ref[0] * wt_ref[:, 0:1] + gc_ref[1] * wt_ref[:, 1:2]
    out = lax.dot_general(comb.astype(jnp.bfloat16),
                          wo_ref[...].astype(jnp.bfloat16),
                          (((1,), (1,)), ((), ())),
                          preferred_element_type=jnp.float32)
    out = out + bo_ref[...]
    res = x_ref[...] + out
    ms = jnp.mean(res * res, axis=1, keepdims=True)
    y_ref[...] = g_ref[...] * (res * lax.rsqrt(ms + EPS))


def _output(x2d, gc, wt, Wo, bo, g):
    return pl.pallas_call(
        _output_body,
        grid=(TOKENS // OUT_TM,),
        in_specs=[
            pl.BlockSpec((OUT_TM, DIM), lambda i: (i, 0)),
            pl.BlockSpec((TOP_K, OUT_TM, DIM), lambda i: (0, i, 0)),
            pl.BlockSpec((OUT_TM, TOP_K), lambda i: (i, 0)),
            pl.BlockSpec((DIM, DIM), lambda i: (0, 0)),
            pl.BlockSpec((1, DIM), lambda i: (0, 0)),
            pl.BlockSpec((1, DIM), lambda i: (0, 0)),
        ],
        out_specs=pl.BlockSpec((OUT_TM, DIM), lambda i: (i, 0)),
        out_shape=jax.ShapeDtypeStruct((TOKENS, DIM), jnp.float32),
    )(x2d, gc, wt, Wo, bo.reshape(1, DIM), g.reshape(1, DIM))


def kernel(x, Wr, br, We, be, Wo, bo, g):
    x2d = x.reshape(TOKENS, DIM)
    pos, w, te = _routing(x2d, Wr, br)
    pos_flat = pos.reshape(SLOTS)
    te_flat = te.reshape(128)
    wt = w.T
    xg = _sc_dispatch(x2d, pos_flat)
    yg = _expert_matmuls(te_flat, xg, We, be)
    gc = _sc_combine(yg, pos_flat)
    y = _output(x2d, gc.reshape(TOP_K, TOKENS, DIM), wt, Wo, bo, g)
    return y.reshape(x.shape)


# speedup vs baseline: 1.2967x; 1.2967x over previous
import functools

import jax
import jax.numpy as jnp
from jax import lax
from jax.experimental import pallas as pl
from jax.experimental.pallas import tpu as pltpu
from jax.experimental.pallas import tpu_sc as plsc

DIM = 1024
NUM_EXPERTS = 8
TOP_K = 2
TOKENS = 4096
TILE = 128
NUM_TILES = 72
NUM_SLOTS = NUM_TILES * TILE
SLOTS = TOKENS * TOP_K
NUM_WORKERS = 32
SLOTS_PER_WORKER = SLOTS // NUM_WORKERS
CHUNK = 64
EPS = 1e-6
OUT_TM = 512


def _lane_cumsum(a):
    n = a.shape[-1]
    d = 1
    while d < n:
        shifted = jnp.concatenate(
            [jnp.zeros(a.shape[:-1] + (d,), a.dtype), a[..., :-d]], axis=-1)
        a = a + shifted
        d *= 2
    return a


def _routing_body(x_ref, wr_ref, br_ref, pos_ref, w_ref, te_ref):
    x = x_ref[...]
    lt = lax.dot_general(wr_ref[...], x, (((1,), (1,)), ((), ())),
                         precision=lax.Precision.HIGHEST,
                         preferred_element_type=jnp.float32)
    lt = lt + br_ref[...]
    riota = lax.broadcasted_iota(jnp.int32, lt.shape, 0)
    m1 = jnp.max(lt, axis=0, keepdims=True)
    i1 = jnp.min(jnp.where(lt == m1, riota, NUM_EXPERTS), axis=0,
                 keepdims=True)
    sel1 = riota == i1
    lt2 = jnp.where(sel1, -1e30, lt)
    m2 = jnp.max(lt2, axis=0, keepdims=True)
    i2 = jnp.min(jnp.where(lt2 == m2, riota, NUM_EXPERTS), axis=0,
                 keepdims=True)
    sel2 = riota == i2
    w1 = jax.nn.sigmoid(m1 - m2)
    w_ref[...] = jnp.concatenate([w1, 1.0 - w1], axis=0)
    a1 = sel1.astype(jnp.int32)
    a2 = sel2.astype(jnp.int32)
    c1 = _lane_cumsum(a1)
    c2 = _lane_cumsum(a2)
    tot1 = c1[:, TOKENS - 1:]
    counts = tot1 + c2[:, TOKENS - 1:]
    ntiles = (counts + (TILE - 1)) >> 7
    inc = ntiles
    for d in (1, 2, 4):
        inc = inc + jnp.concatenate(
            [jnp.zeros((d, 1), jnp.int32), inc[:-d]], axis=0)
    toff = inc - ntiles
    base = toff * TILE
    p0 = base + (c1 - a1)
    p1 = base + tot1 + (c2 - a2)
    pos0 = jnp.sum(a1 * p0, axis=0, keepdims=True)
    pos1 = jnp.sum(a2 * p1, axis=0, keepdims=True)
    pos_ref[...] = jnp.concatenate([pos0, pos1], axis=0)
    jiota = lax.broadcasted_iota(jnp.int32, (NUM_EXPERTS, 128), 1)
    te_ref[...] = jnp.sum((jiota >= toff).astype(jnp.int32), axis=0,
                          keepdims=True) - 1


def _routing(x2d, Wr, br):
    return pl.pallas_call(
        _routing_body,
        out_shape=(
            jax.ShapeDtypeStruct((TOP_K, TOKENS), jnp.int32),
            jax.ShapeDtypeStruct((TOP_K, TOKENS), jnp.float32),
            jax.ShapeDtypeStruct((1, 128), jnp.int32),
        ),
    )(x2d, Wr, br.reshape(NUM_EXPERTS, 1))


@functools.cache
def _sc_kernels():
    mesh = plsc.VectorSubcoreMesh(core_axis_name="c", subcore_axis_name="s")
    scratch = [
        pltpu.VMEM((CHUNK,), jnp.int32),
        pltpu.VMEM((CHUNK, DIM), jnp.float32),
        pltpu.SemaphoreType.DMA,
    ]

    @functools.partial(
        pl.kernel,
        out_type=jax.ShapeDtypeStruct((NUM_SLOTS, DIM), jnp.float32),
        mesh=mesh,
        scratch_types=scratch,
    )
    def sc_dispatch(x_hbm, pos_hbm, xg_hbm, pos_v, rows_v, sem):
        wid = lax.axis_index("s") * 2 + lax.axis_index("c")

        @pl.loop(0, SLOTS_PER_WORKER // CHUNK)
        def _(ci):
            s_base = wid * SLOTS_PER_WORKER + ci * CHUNK
            t_base = lax.rem(s_base, TOKENS)
            pltpu.sync_copy(pos_hbm.at[pl.ds(s_base, CHUNK)], pos_v)
            pltpu.sync_copy(x_hbm.at[pl.ds(t_base, CHUNK)], rows_v)
            pltpu.async_copy(rows_v, xg_hbm.at[pos_v], sem).wait()

    @functools.partial(
        pl.kernel,
        out_type=jax.ShapeDtypeStruct((SLOTS, DIM), jnp.float32),
        mesh=mesh,
        scratch_types=scratch,
    )
    def sc_combine(yg_hbm, pos_hbm, gc_hbm, pos_v, rows_v, sem):
        wid = lax.axis_index("s") * 2 + lax.axis_index("c")

        @pl.loop(0, SLOTS_PER_WORKER // CHUNK)
        def _(ci):
            s_base = wid * SLOTS_PER_WORKER + ci * CHUNK
            pltpu.sync_copy(pos_hbm.at[pl.ds(s_base, CHUNK)], pos_v)
            pltpu.async_copy(yg_hbm.at[pos_v], rows_v, sem).wait()
            pltpu.sync_copy(rows_v, gc_hbm.at[pl.ds(s_base, CHUNK)])

    return sc_dispatch, sc_combine


def _sc_dispatch(x2d, pos_flat):
    return _sc_kernels()[0](x2d, pos_flat)


def _sc_combine(yg, pos_flat):
    return _sc_kernels()[1](yg, pos_flat)


def _expert_body(te_ref, xg_ref, we_ref, be_ref, yg_ref):
    xb = xg_ref[...].astype(jnp.bfloat16)
    wb = we_ref[0].astype(jnp.bfloat16)
    h = lax.dot_general(xb, wb, (((1,), (1,)), ((), ())),
                        preferred_element_type=jnp.float32)
    h = h + be_ref[0]
    yg_ref[...] = h * jax.nn.sigmoid(h)


def _expert_matmuls(te, xg, We, be):
    grid_spec = pltpu.PrefetchScalarGridSpec(
        num_scalar_prefetch=1,
        grid=(NUM_TILES,),
        in_specs=[
            pl.BlockSpec((TILE, DIM), lambda j, te: (j, 0)),
            pl.BlockSpec((1, DIM, DIM), lambda j, te: (te[j], 0, 0)),
            pl.BlockSpec((1, 1, DIM), lambda j, te: (te[j], 0, 0)),
        ],
        out_specs=pl.BlockSpec((TILE, DIM), lambda j, te: (j, 0)),
    )
    return pl.pallas_call(
        _expert_body,
        grid_spec=grid_spec,
        out_shape=jax.ShapeDtypeStruct((NUM_SLOTS, DIM), jnp.float32),
    )(te, xg, We, be.reshape(NUM_EXPERTS, 1, DIM))


def _output_body(x_ref, gc_ref, wt_ref, wo_ref, bo_ref, g_ref, y_ref):
    comb = gc_ref[0] * wt_ref[:, 0:1] + gc_ref[1] * wt_ref[:, 1:2]
    out = lax.dot_general(comb.astype(jnp.bfloat16),
                          wo_ref[...].astype(jnp.bfloat16),
                          (((1,), (1,)), ((), ())),
                          preferred_element_type=jnp.float32)
    out = out + bo_ref[...]
    res = x_ref[...] + out
    ms = jnp.mean(res * res, axis=1, keepdims=True)
    y_ref[...] = g_ref[...] * (res * lax.rsqrt(ms + EPS))


def _output(x2d, gc, wt, Wo, bo, g):
    return pl.pallas_call(
        _output_body,
        grid=(TOKENS // OUT_TM,),
        in_specs=[
            pl.BlockSpec((OUT_TM, DIM), lambda i: (i, 0)),
            pl.BlockSpec((TOP_K, OUT_TM, DIM), lambda i: (0, i, 0)),
            pl.BlockSpec((OUT_TM, TOP_K), lambda i: (i, 0)),
            pl.BlockSpec((DIM, DIM), lambda i: (0, 0)),
            pl.BlockSpec((1, DIM), lambda i: (0, 0)),
            pl.BlockSpec((1, DIM), lambda i: (0, 0)),
        ],
        out_specs=pl.BlockSpec((OUT_TM, DIM), lambda i: (i, 0)),
        out_shape=jax.ShapeDtypeStruct((TOKENS, DIM), jnp.float32),
    )(x2d, gc, wt, Wo, bo.reshape(1, DIM), g.reshape(1, DIM))


def kernel(x, Wr, br, We, be, Wo, bo, g):
    x2d = x.reshape(TOKENS, DIM)
    pos, w, te = _routing(x2d, Wr, br)
    pos_flat = pos.reshape(SLOTS)
    te_flat = te.reshape(128)
    wt = w.T
    xg = _sc_dispatch(x2d, pos_flat)
    yg = _expert_matmuls(te_flat, xg, We, be)
    gc = _sc_combine(yg, pos_flat)
    y = _output(x2d, gc.reshape(TOP_K, TOKENS, DIM), wt, Wo, bo, g)
    return y.reshape(x.shape)


# speedup vs baseline: 1.4341x; 1.1060x over previous
import functools

import jax
import jax.numpy as jnp
from jax import lax
from jax.experimental import pallas as pl
from jax.experimental.pallas import tpu as pltpu
from jax.experimental.pallas import tpu_sc as plsc

DIM = 1024
NUM_EXPERTS = 8
TOP_K = 2
TOKENS = 4096
TILE = 256
NUM_TILES = 40
NUM_SLOTS = NUM_TILES * TILE
SLOTS = TOKENS * TOP_K
NUM_WORKERS = 32
SLOTS_PER_WORKER = SLOTS // NUM_WORKERS
CHUNK = 64
EPS = 1e-6
OUT_TM = 512


def _lane_cumsum(a):
    n = a.shape[-1]
    d = 1
    while d < n:
        shifted = jnp.concatenate(
            [jnp.zeros(a.shape[:-1] + (d,), a.dtype), a[..., :-d]], axis=-1)
        a = a + shifted
        d *= 2
    return a


def _routing_body(x_ref, wr_ref, br_ref, pos_ref, w_ref, te_ref):
    x = x_ref[...]
    lt = lax.dot_general(wr_ref[...], x, (((1,), (1,)), ((), ())),
                         precision=lax.Precision.HIGHEST,
                         preferred_element_type=jnp.float32)
    lt = lt + br_ref[...]
    riota = lax.broadcasted_iota(jnp.int32, lt.shape, 0)
    m1 = jnp.max(lt, axis=0, keepdims=True)
    i1 = jnp.min(jnp.where(lt == m1, riota, NUM_EXPERTS), axis=0,
                 keepdims=True)
    sel1 = riota == i1
    lt2 = jnp.where(sel1, -1e30, lt)
    m2 = jnp.max(lt2, axis=0, keepdims=True)
    i2 = jnp.min(jnp.where(lt2 == m2, riota, NUM_EXPERTS), axis=0,
                 keepdims=True)
    sel2 = riota == i2
    w1 = jax.nn.sigmoid(m1 - m2)
    w_ref[...] = jnp.concatenate([w1, 1.0 - w1], axis=0)
    a1 = sel1.astype(jnp.int32)
    a2 = sel2.astype(jnp.int32)
    c1 = _lane_cumsum(a1)
    c2 = _lane_cumsum(a2)
    tot1 = c1[:, TOKENS - 1:]
    counts = tot1 + c2[:, TOKENS - 1:]
    ntiles = (counts + (TILE - 1)) >> 8
    inc = ntiles
    for d in (1, 2, 4):
        inc = inc + jnp.concatenate(
            [jnp.zeros((d, 1), jnp.int32), inc[:-d]], axis=0)
    toff = inc - ntiles
    base = toff * TILE
    p0 = base + (c1 - a1)
    p1 = base + tot1 + (c2 - a2)
    pos0 = jnp.sum(a1 * p0, axis=0, keepdims=True)
    pos1 = jnp.sum(a2 * p1, axis=0, keepdims=True)
    pos_ref[...] = jnp.concatenate([pos0, pos1], axis=0)
    jiota = lax.broadcasted_iota(jnp.int32, (NUM_EXPERTS, 128), 1)
    te_ref[...] = jnp.sum((jiota >= toff).astype(jnp.int32), axis=0,
                          keepdims=True) - 1


def _routing(x2d, Wr, br):
    return pl.pallas_call(
        _routing_body,
        out_shape=(
            jax.ShapeDtypeStruct((TOP_K, TOKENS), jnp.int32),
            jax.ShapeDtypeStruct((TOP_K, TOKENS), jnp.float32),
            jax.ShapeDtypeStruct((1, 128), jnp.int32),
        ),
    )(x2d, Wr, br.reshape(NUM_EXPERTS, 1))


@functools.cache
def _sc_kernels():
    mesh = plsc.VectorSubcoreMesh(core_axis_name="c", subcore_axis_name="s")
    scratch = [
        pltpu.VMEM((CHUNK,), jnp.int32),
        pltpu.VMEM((CHUNK, DIM), jnp.float32),
        pltpu.SemaphoreType.DMA,
    ]

    @functools.partial(
        pl.kernel,
        out_type=jax.ShapeDtypeStruct((NUM_SLOTS, DIM), jnp.float32),
        mesh=mesh,
        scratch_types=scratch,
    )
    def sc_dispatch(x_hbm, pos_hbm, xg_hbm, pos_v, rows_v, sem):
        wid = lax.axis_index("s") * 2 + lax.axis_index("c")

        @pl.loop(0, SLOTS_PER_WORKER // CHUNK)
        def _(ci):
            s_base = wid * SLOTS_PER_WORKER + ci * CHUNK
            t_base = lax.rem(s_base, TOKENS)
            pltpu.sync_copy(pos_hbm.at[pl.ds(s_base, CHUNK)], pos_v)
            pltpu.sync_copy(x_hbm.at[pl.ds(t_base, CHUNK)], rows_v)
            pltpu.async_copy(rows_v, xg_hbm.at[pos_v], sem).wait()

    @functools.partial(
        pl.kernel,
        out_type=jax.ShapeDtypeStruct((SLOTS, DIM), jnp.float32),
        mesh=mesh,
        scratch_types=scratch,
    )
    def sc_combine(yg_hbm, pos_hbm, gc_hbm, pos_v, rows_v, sem):
        wid = lax.axis_index("s") * 2 + lax.axis_index("c")

        @pl.loop(0, SLOTS_PER_WORKER // CHUNK)
        def _(ci):
            s_base = wid * SLOTS_PER_WORKER + ci * CHUNK
            pltpu.sync_copy(pos_hbm.at[pl.ds(s_base, CHUNK)], pos_v)
            pltpu.async_copy(yg_hbm.at[pos_v], rows_v, sem).wait()
            pltpu.sync_copy(rows_v, gc_hbm.at[pl.ds(s_base, CHUNK)])

    return sc_dispatch, sc_combine


def _sc_dispatch(x2d, pos_flat):
    return _sc_kernels()[0](x2d, pos_flat)


def _sc_combine(yg, pos_flat):
    return _sc_kernels()[1](yg, pos_flat)


def _expert_body(te_ref, xg_ref, we_ref, be_ref, yg_ref):
    h = lax.dot_general(xg_ref[...].astype(jnp.bfloat16), we_ref[0],
                        (((1,), (1,)), ((), ())),
                        preferred_element_type=jnp.float32)
    h = h + be_ref[0]
    yg_ref[...] = h * jax.nn.sigmoid(h)


def _expert_matmuls(te, xg, We, be):
    grid_spec = pltpu.PrefetchScalarGridSpec(
        num_scalar_prefetch=1,
        grid=(NUM_TILES,),
        in_specs=[
            pl.BlockSpec((TILE, DIM), lambda j, te: (j, 0)),
            pl.BlockSpec((1, DIM, DIM), lambda j, te: (te[j], 0, 0)),
            pl.BlockSpec((1, 1, DIM), lambda j, te: (te[j], 0, 0)),
        ],
        out_specs=pl.BlockSpec((TILE, DIM), lambda j, te: (j, 0)),
    )
    return pl.pallas_call(
        _expert_body,
        grid_spec=grid_spec,
        out_shape=jax.ShapeDtypeStruct((NUM_SLOTS, DIM), jnp.float32),
    )(te, xg, We, be.reshape(NUM_EXPERTS, 1, DIM))


def _output_body(x_ref, gc_ref, wt_ref, wo_ref, bo_ref, g_ref, y_ref):
    comb = gc_ref[0] * wt_ref[:, 0:1] + gc_ref[1] * wt_ref[:, 1:2]
    out = lax.dot_general(comb.astype(jnp.bfloat16), wo_ref[...],
                          (((1,), (1,)), ((), ())),
                          preferred_element_type=jnp.float32)
    out = out + bo_ref[...]
    res = x_ref[...] + out
    ms = jnp.mean(res * res, axis=1, keepdims=True)
    y_ref[...] = g_ref[...] * (res * lax.rsqrt(ms + EPS))


def _output(x2d, gc, wt, Wo, bo, g):
    return pl.pallas_call(
        _output_body,
        grid=(TOKENS // OUT_TM,),
        in_specs=[
            pl.BlockSpec((OUT_TM, DIM), lambda i: (i, 0)),
            pl.BlockSpec((TOP_K, OUT_TM, DIM), lambda i: (0, i, 0)),
            pl.BlockSpec((OUT_TM, TOP_K), lambda i: (i, 0)),
            pl.BlockSpec((DIM, DIM), lambda i: (0, 0)),
            pl.BlockSpec((1, DIM), lambda i: (0, 0)),
            pl.BlockSpec((1, DIM), lambda i: (0, 0)),
        ],
        out_specs=pl.BlockSpec((OUT_TM, DIM), lambda i: (i, 0)),
        out_shape=jax.ShapeDtypeStruct((TOKENS, DIM), jnp.float32),
    )(x2d, gc, wt, Wo, bo.reshape(1, DIM), g.reshape(1, DIM))


def kernel(x, Wr, br, We, be, Wo, bo, g):
    x2d = x.reshape(TOKENS, DIM)
    pos, w, te = _routing(x2d, Wr, br)
    pos_flat = pos.reshape(SLOTS)
    te_flat = te.reshape(128)
    wt = w.T
    xg = _sc_dispatch(x2d, pos_flat)
    yg = _expert_matmuls(te_flat, xg, We.astype(jnp.bfloat16), be)
    gc = _sc_combine(yg, pos_flat)
    y = _output(x2d, gc.reshape(TOP_K, TOKENS, DIM), wt,
                Wo.astype(jnp.bfloat16), bo, g)
    return y.reshape(x.shape)


# speedup vs baseline: 1.5211x; 1.0607x over previous
import functools

import jax
import jax.numpy as jnp
from jax import lax
from jax.experimental import pallas as pl
from jax.experimental.pallas import tpu as pltpu
from jax.experimental.pallas import tpu_sc as plsc

DIM = 1024
NUM_EXPERTS = 8
TOP_K = 2
TOKENS = 4096
TILE = 256
NUM_TILES = 40
NUM_SLOTS = NUM_TILES * TILE
SLOTS = TOKENS * TOP_K
NUM_WORKERS = 32
SLOTS_PER_WORKER = SLOTS // NUM_WORKERS
CHUNK = 64
EPS = 1e-6
OUT_TM = 512


def _lane_cumsum(a):
    n = a.shape[-1]
    d = 1
    while d < n:
        shifted = jnp.concatenate(
            [jnp.zeros(a.shape[:-1] + (d,), a.dtype), a[..., :-d]], axis=-1)
        a = a + shifted
        d *= 2
    return a


def _routing_body(x_ref, wr_ref, br_ref, pos_ref, w_ref, te_ref):
    x = x_ref[...]
    lt = lax.dot_general(wr_ref[...].astype(jnp.bfloat16),
                         x.astype(jnp.bfloat16), (((1,), (1,)), ((), ())),
                         preferred_element_type=jnp.float32)
    lt = lt + br_ref[...]
    riota = lax.broadcasted_iota(jnp.int32, lt.shape, 0)
    m1 = jnp.max(lt, axis=0, keepdims=True)
    i1 = jnp.min(jnp.where(lt == m1, riota, NUM_EXPERTS), axis=0,
                 keepdims=True)
    sel1 = riota == i1
    lt2 = jnp.where(sel1, -1e30, lt)
    m2 = jnp.max(lt2, axis=0, keepdims=True)
    i2 = jnp.min(jnp.where(lt2 == m2, riota, NUM_EXPERTS), axis=0,
                 keepdims=True)
    sel2 = riota == i2
    w1 = jax.nn.sigmoid(m1 - m2)
    w_ref[...] = jnp.concatenate([w1, 1.0 - w1], axis=0)
    a1 = sel1.astype(jnp.int32)
    a2 = sel2.astype(jnp.int32)
    c1 = _lane_cumsum(a1)
    c2 = _lane_cumsum(a2)
    tot1 = c1[:, TOKENS - 1:]
    counts = tot1 + c2[:, TOKENS - 1:]
    ntiles = (counts + (TILE - 1)) >> 8
    inc = ntiles
    for d in (1, 2, 4):
        inc = inc + jnp.concatenate(
            [jnp.zeros((d, 1), jnp.int32), inc[:-d]], axis=0)
    toff = inc - ntiles
    base = toff * TILE
    p0 = base + (c1 - a1)
    p1 = base + tot1 + (c2 - a2)
    pos0 = jnp.sum(a1 * p0, axis=0, keepdims=True)
    pos1 = jnp.sum(a2 * p1, axis=0, keepdims=True)
    pos_ref[...] = jnp.concatenate([pos0, pos1], axis=0)
    jiota = lax.broadcasted_iota(jnp.int32, (NUM_EXPERTS, 128), 1)
    te_ref[...] = jnp.sum((jiota >= toff).astype(jnp.int32), axis=0,
                          keepdims=True) - 1


def _routing(x2d, Wr, br):
    return pl.pallas_call(
        _routing_body,
        out_shape=(
            jax.ShapeDtypeStruct((TOP_K, TOKENS), jnp.int32),
            jax.ShapeDtypeStruct((TOP_K, TOKENS), jnp.float32),
            jax.ShapeDtypeStruct((1, 128), jnp.int32),
        ),
    )(x2d, Wr, br.reshape(NUM_EXPERTS, 1))


@functools.cache
def _sc_kernels():
    mesh = plsc.VectorSubcoreMesh(core_axis_name="c", subcore_axis_name="s")
    scratch = [
        pltpu.VMEM((CHUNK,), jnp.int32),
        pltpu.VMEM((CHUNK, DIM), jnp.float32),
        pltpu.SemaphoreType.DMA,
    ]

    @functools.partial(
        pl.kernel,
        out_type=jax.ShapeDtypeStruct((NUM_SLOTS, DIM), jnp.float32),
        mesh=mesh,
        scratch_types=scratch,
    )
    def sc_dispatch(x_hbm, pos_hbm, xg_hbm, pos_v, rows_v, sem):
        wid = lax.axis_index("s") * 2 + lax.axis_index("c")

        @pl.loop(0, SLOTS_PER_WORKER // CHUNK)
        def _(ci):
            s_base = wid * SLOTS_PER_WORKER + ci * CHUNK
            t_base = lax.rem(s_base, TOKENS)
            pltpu.sync_copy(pos_hbm.at[pl.ds(s_base, CHUNK)], pos_v)
            pltpu.sync_copy(x_hbm.at[pl.ds(t_base, CHUNK)], rows_v)
            pltpu.async_copy(rows_v, xg_hbm.at[pos_v], sem).wait()

    @functools.partial(
        pl.kernel,
        out_type=jax.ShapeDtypeStruct((SLOTS, DIM), jnp.float32),
        mesh=mesh,
        scratch_types=scratch,
    )
    def sc_combine(yg_hbm, pos_hbm, gc_hbm, pos_v, rows_v, sem):
        wid = lax.axis_index("s") * 2 + lax.axis_index("c")

        @pl.loop(0, SLOTS_PER_WORKER // CHUNK)
        def _(ci):
            s_base = wid * SLOTS_PER_WORKER + ci * CHUNK
            pltpu.sync_copy(pos_hbm.at[pl.ds(s_base, CHUNK)], pos_v)
            pltpu.async_copy(yg_hbm.at[pos_v], rows_v, sem).wait()
            pltpu.sync_copy(rows_v, gc_hbm.at[pl.ds(s_base, CHUNK)])

    return sc_dispatch, sc_combine


def _sc_dispatch(x2d, pos_flat):
    return _sc_kernels()[0](x2d, pos_flat)


def _sc_combine(yg, pos_flat):
    return _sc_kernels()[1](yg, pos_flat)


def _expert_body(te_ref, xg_ref, we_ref, be_ref, yg_ref):
    h = lax.dot_general(xg_ref[...].astype(jnp.bfloat16), we_ref[0],
                        (((1,), (1,)), ((), ())),
                        preferred_element_type=jnp.float32)
    h = h + be_ref[0]
    yg_ref[...] = h * jax.nn.sigmoid(h)


def _expert_matmuls(te, xg, We, be):
    grid_spec = pltpu.PrefetchScalarGridSpec(
        num_scalar_prefetch=1,
        grid=(NUM_TILES,),
        in_specs=[
            pl.BlockSpec((TILE, DIM), lambda j, te: (j, 0)),
            pl.BlockSpec((1, DIM, DIM), lambda j, te: (te[j], 0, 0)),
            pl.BlockSpec((1, 1, DIM), lambda j, te: (te[j], 0, 0)),
        ],
        out_specs=pl.BlockSpec((TILE, DIM), lambda j, te: (j, 0)),
    )
    return pl.pallas_call(
        _expert_body,
        grid_spec=grid_spec,
        out_shape=jax.ShapeDtypeStruct((NUM_SLOTS, DIM), jnp.float32),
    )(te, xg, We, be.reshape(NUM_EXPERTS, 1, DIM))


def _output_body(x_ref, gc_ref, wt_ref, wo_ref, bo_ref, g_ref, y_ref):
    comb = gc_ref[0] * wt_ref[:, 0:1] + gc_ref[1] * wt_ref[:, 1:2]
    out = lax.dot_general(comb.astype(jnp.bfloat16), wo_ref[...],
                          (((1,), (1,)), ((), ())),
                          preferred_element_type=jnp.float32)
    out = out + bo_ref[...]
    res = x_ref[...] + out
    ms = jnp.mean(res * res, axis=1, keepdims=True)
    y_ref[...] = g_ref[...] * (res * lax.rsqrt(ms + EPS))


def _output(x2d, gc, wt, Wo, bo, g):
    return pl.pallas_call(
        _output_body,
        grid=(TOKENS // OUT_TM,),
        in_specs=[
            pl.BlockSpec((OUT_TM, DIM), lambda i: (i, 0)),
            pl.BlockSpec((TOP_K, OUT_TM, DIM), lambda i: (0, i, 0)),
            pl.BlockSpec((OUT_TM, TOP_K), lambda i: (i, 0)),
            pl.BlockSpec((DIM, DIM), lambda i: (0, 0)),
            pl.BlockSpec((1, DIM), lambda i: (0, 0)),
            pl.BlockSpec((1, DIM), lambda i: (0, 0)),
        ],
        out_specs=pl.BlockSpec((OUT_TM, DIM), lambda i: (i, 0)),
        out_shape=jax.ShapeDtypeStruct((TOKENS, DIM), jnp.float32),
    )(x2d, gc, wt, Wo, bo.reshape(1, DIM), g.reshape(1, DIM))


def kernel(x, Wr, br, We, be, Wo, bo, g):
    x2d = x.reshape(TOKENS, DIM)
    pos, w, te = _routing(x2d, Wr, br)
    pos_flat = pos.reshape(SLOTS)
    te_flat = te.reshape(128)
    wt = w.T
    xg = _sc_dispatch(x2d, pos_flat)
    yg = _expert_matmuls(te_flat, xg, We.astype(jnp.bfloat16), be)
    gc = _sc_combine(yg, pos_flat)
    y = _output(x2d, gc.reshape(TOP_K, TOKENS, DIM), wt,
                Wo.astype(jnp.bfloat16), bo, g)
    return y.reshape(x.shape)
